# baseline (speedup 1.0000x reference)
import functools

import jax
import jax.numpy as jnp
from jax import lax
from jax.experimental import pallas as pl
from jax.experimental.pallas import tpu as pltpu
from jax.experimental.pallas import tpu_sc as plsc

NUM_ROWS = 4096 * 200
DIM = 64
NC = 2
NS = 16
NW = NC * NS
ROWS_PER_W = NUM_ROWS // NW
CHUNK = 128
N_CHUNKS = ROWS_PER_W // CHUNK


def _emb_body(idx_hbm, w_hbm, out_hbm, idx_v, rows_v, gsem):
    wid = lax.axis_index("s") * NC + lax.axis_index("c")
    pltpu.sync_copy(idx_hbm.at[wid], idx_v)

    def body(j, carry):
        pltpu.async_copy(w_hbm.at[idx_v.at[j]], rows_v, gsem).wait()
        pltpu.sync_copy(rows_v, out_hbm.at[wid, j])
        return carry

    lax.fori_loop(0, N_CHUNKS, body, 0)


@jax.jit
def _embedding_lookup(idx, weight):
    mesh = plsc.VectorSubcoreMesh(core_axis_name="c", subcore_axis_name="s")
    k = functools.partial(
        pl.kernel,
        mesh=mesh,
        out_type=jax.ShapeDtypeStruct((NW, N_CHUNKS, CHUNK, DIM), jnp.float32),
        scratch_types=[
            pltpu.VMEM((N_CHUNKS, CHUNK), jnp.int32),
            pltpu.VMEM((CHUNK, DIM), jnp.float32),
            pltpu.SemaphoreType.DMA,
        ],
        compiler_params=pltpu.CompilerParams(use_tc_tiling_on_sc=False),
    )(_emb_body)
    return k(idx, weight)


def kernel(x, weight):
    idx = x.reshape(NW, N_CHUNKS, CHUNK)
    out = _embedding_lookup(idx, weight)
    return out.reshape(x.shape[0], x.shape[1], DIM)


# speedup vs baseline: 1.1152x; 1.1152x over previous
import functools

import jax
import jax.numpy as jnp
from jax import lax
from jax.experimental import pallas as pl
from jax.experimental.pallas import tpu as pltpu
from jax.experimental.pallas import tpu_sc as plsc

NUM_ROWS = 4096 * 200
DIM = 64
NC = 2
NS = 16
NW = NC * NS
ROWS_PER_W = NUM_ROWS // NW
CHUNK = 128
N_CHUNKS = ROWS_PER_W // CHUNK
NBUF = 4
N_GROUPS = N_CHUNKS // NBUF


def _emb_body(idx_hbm, w_hbm, out_hbm, idx_v, rows_v, gsem0, gsem1, ssem0, ssem1):
    wid = lax.axis_index("s") * NC + lax.axis_index("c")
    pltpu.sync_copy(idx_hbm.at[wid], idx_v)

    def fire_gathers(g, h, gsem):
        @pl.when(g < N_GROUPS)
        def _():
            for b in range(NBUF):
                c = g * NBUF + b
                pltpu.async_copy(w_hbm.at[idx_v.at[c]], rows_v.at[h, b], gsem)

    def drain_gathers_fire_stores(g, h, gsem, ssem):
        for b in range(NBUF):
            c = g * NBUF + b
            pltpu.make_async_copy(w_hbm.at[idx_v.at[c]], rows_v.at[h, b], gsem).wait()
            pltpu.async_copy(rows_v.at[h, b], out_hbm.at[wid, c], ssem)

    def drain_stores(g, h, ssem):
        for b in range(NBUF):
            c = g * NBUF + b
            pltpu.make_async_copy(rows_v.at[h, b], out_hbm.at[wid, c], ssem).wait()

    fire_gathers(0, 0, gsem0)
    fire_gathers(1, 1, gsem1)

    def body(t, carry):
        g0 = 2 * t
        g1 = 2 * t + 1
        drain_gathers_fire_stores(g0, 0, gsem0, ssem0)
        drain_gathers_fire_stores(g1, 1, gsem1, ssem1)
        drain_stores(g0, 0, ssem0)
        fire_gathers(g0 + 2, 0, gsem0)
        drain_stores(g1, 1, ssem1)
        fire_gathers(g1 + 2, 1, gsem1)
        return carry

    lax.fori_loop(0, N_GROUPS // 2, body, 0)


@jax.jit
def _embedding_lookup(idx, weight):
    mesh = plsc.VectorSubcoreMesh(core_axis_name="c", subcore_axis_name="s")
    k = functools.partial(
        pl.kernel,
        mesh=mesh,
        out_type=jax.ShapeDtypeStruct((NW, N_CHUNKS, CHUNK, DIM), jnp.float32),
        scratch_types=[
            pltpu.VMEM((N_CHUNKS, CHUNK), jnp.int32),
            pltpu.VMEM((2, NBUF, CHUNK, DIM), jnp.float32),
            pltpu.SemaphoreType.DMA,
            pltpu.SemaphoreType.DMA,
            pltpu.SemaphoreType.DMA,
            pltpu.SemaphoreType.DMA,
        ],
        compiler_params=pltpu.CompilerParams(use_tc_tiling_on_sc=False),
    )(_emb_body)
    return k(idx, weight)


def kernel(x, weight):
    idx = x.reshape(NW, N_CHUNKS, CHUNK)
    out = _embedding_lookup(idx, weight)
    return out.reshape(x.shape[0], x.shape[1], DIM)
